# Initial kernel scaffold; baseline (speedup 1.0000x reference)
import jax
import jax.numpy as jnp
from jax.experimental import pallas as pl


def kernel(x, pe_k, pe_v, comp, bases, root, bias):
    raise NotImplementedError("write your pallas kernel here")


# speedup vs baseline: 13.8859x; 13.8859x over previous
import functools

import jax
import jax.numpy as jnp
from jax import lax
from jax.experimental import pallas as pl
from jax.experimental.pallas import tpu as pltpu
from jax.experimental.pallas import tpu_sc as plsc

_SLEN = 256
_DIM = 512
_PDIM = 64

_NC = 2
_NS = 16
_NW = _NC * _NS
_TPW = 16
_WIN = 272
_HEAD = 264


def _out_body(x_ref, comp_ref, bases_ref, root_ref, bias_ref, o_ref):
    x = x_ref[...]
    w0 = root_ref[...]
    w1 = jnp.zeros((_DIM, _DIM), jnp.float32)
    for b in range(4):
        w0 = w0 + comp_ref[0, b] * bases_ref[b]
        w1 = w1 + comp_ref[1, b] * bases_ref[b]
    row = lax.broadcasted_iota(jnp.int32, (_SLEN, _SLEN), 0)
    col = lax.broadcasted_iota(jnp.int32, (_SLEN, _SLEN), 1)
    tri = (col < row).astype(jnp.float32)
    s = jnp.dot(tri, x, preferred_element_type=jnp.float32)
    n = lax.broadcasted_iota(jnp.int32, (_SLEN, 1), 0).astype(jnp.float32)
    p = s / jnp.maximum(n, 1.0)
    o_ref[...] = (
        jnp.dot(x, w0, preferred_element_type=jnp.float32)
        + jnp.dot(p, w1, preferred_element_type=jnp.float32)
        + bias_ref[...]
    )


def _tc_out(x, comp, bases, root, bias):
    return pl.pallas_call(
        _out_body,
        out_shape=jax.ShapeDtypeStruct((_SLEN, _DIM), jnp.float32),
        in_specs=[
            pl.BlockSpec(memory_space=pltpu.VMEM),
            pl.BlockSpec(memory_space=pltpu.SMEM),
            pl.BlockSpec(memory_space=pltpu.VMEM),
            pl.BlockSpec(memory_space=pltpu.VMEM),
            pl.BlockSpec(memory_space=pltpu.VMEM),
        ],
        out_specs=pl.BlockSpec(memory_space=pltpu.VMEM),
    )(x, comp, bases, root, bias.reshape(1, _DIM))


def _rel_body(pe_k_hbm, pe_v_hbm, outk_hbm, outv_hbm, ph, w, sem):
    wid = lax.axis_index("s") * _NC + lax.axis_index("c")
    t0 = (wid % _NS) * _TPW

    def do_table(pe_hbm, out_hbm):
        pltpu.async_copy(pe_hbm.at[pl.ds(0, _HEAD)], ph, sem).wait()

        def rev_row(v, carry):
            src = jnp.maximum(_TPW + t0 - v, 0)
            for c in range(_PDIM // 16):
                w[v, pl.ds(c * 16, 16)] = ph[src, pl.ds(c * 16, 16)]
            return carry

        lax.fori_loop(0, _WIN, rev_row, None)

        h_out = []
        for j in range(_TPW):
            h_out.append(pltpu.async_copy(w.at[pl.ds(_TPW - 1 - j, _SLEN)],
                                          out_hbm.at[t0 + j], sem))
        for h in h_out:
            h.wait()

    @pl.when(wid < _NS)
    def _():
        do_table(pe_k_hbm, outk_hbm)

    @pl.when(wid >= _NS)
    def _():
        do_table(pe_v_hbm, outv_hbm)


def _sc_rel_emb(pe_k, pe_v):
    mesh = plsc.VectorSubcoreMesh(
        core_axis_name="c", subcore_axis_name="s",
        num_cores=_NC, num_subcores=_NS)
    fn = functools.partial(
        pl.kernel,
        out_type=(
            jax.ShapeDtypeStruct((_SLEN, _SLEN, _PDIM), jnp.float32),
            jax.ShapeDtypeStruct((_SLEN, _SLEN, _PDIM), jnp.float32),
        ),
        mesh=mesh,
        scratch_types=[
            pltpu.VMEM((_HEAD, _PDIM), jnp.float32),
            pltpu.VMEM((_WIN, _PDIM), jnp.float32),
            pltpu.SemaphoreType.DMA,
        ],
    )(_rel_body)
    return fn(pe_k, pe_v)


def kernel(x, pe_k, pe_v, comp, bases, root, bias):
    out = _tc_out(x, comp, bases, root, bias)
    rel_emb_k, rel_emb_v = _sc_rel_emb(pe_k, pe_v)
    return out, rel_emb_k, rel_emb_v
